# Initial kernel scaffold; baseline (speedup 1.0000x reference)
import jax
import jax.numpy as jnp
from jax.experimental import pallas as pl


def kernel(rw_t3, params, edge_index, batch, num_graphs):
    raise NotImplementedError("write your pallas kernel here")


# speedup vs baseline: 16.9678x; 16.9678x over previous
import functools
import math

import jax
import jax.numpy as jnp
from jax import lax
from jax.experimental import pallas as pl
from jax.experimental.pallas import tpu as pltpu
from jax.experimental.pallas import tpu_sc as plsc

K = 8
HID = 32
DF = K * HID
CB = 16
NCB = DF // CB
CH = 1024
SUB = 128
NSUB = CH // SUB
NG = 64


def _sc_scatter_wide(z2d, ei, N, Npad, Epad):
    RACC = Npad
    EPW = Epad // 16
    NCHUNK = EPW // CH
    RPT = Npad // 16
    RZT = RPT
    mesh = plsc.VectorSubcoreMesh(core_axis_name="c", subcore_axis_name="s")

    @functools.partial(
        pl.kernel,
        out_type=jax.ShapeDtypeStruct((Npad, NCB, CB), jnp.float32),
        mesh=mesh,
        compiler_params=pltpu.CompilerParams(use_tc_tiling_on_sc=False),
        scratch_types=[
            pltpu.VMEM((CH,), jnp.int32),
            pltpu.VMEM((CH,), jnp.int32),
            pltpu.VMEM((NSUB, SUB), jnp.int32),
            pltpu.VMEM((NSUB, SUB), jnp.int32),
            pltpu.VMEM((CH, CB), jnp.float32),
            pltpu.VMEM_SHARED((RACC, CB), jnp.float32),
            pltpu.SemaphoreType.DMA,
        ],
    )
    def scat(z_hbm, ei_hbm, agg_hbm, row1, col1, gidx, sidx, gbuf,
             acc, sem):
        c = lax.axis_index("c")
        s = lax.axis_index("s")

        zr0 = s * RZT
        wr0 = s * RPT
        e_base = s * EPW

        def one_pass(p, _):
            cbi = 2 * p + c
            def zrow(i, _):
                gbuf[i, :] = jnp.zeros((CB,), jnp.float32)
                return 0
            lax.fori_loop(0, CH, zrow, 0)
            off = 0
            while off < RZT:
                sz = min(CH, RZT - off)
                pltpu.sync_copy(gbuf.at[pl.ds(0, sz)],
                                acc.at[pl.ds(zr0 + off, sz)])
                off += sz
            plsc.subcore_barrier()

            def chunk(g, _):
                e0 = pl.multiple_of(e_base + g * CH, 8)
                pltpu.sync_copy(ei_hbm.at[0, pl.ds(e0, CH)], row1)
                pltpu.sync_copy(ei_hbm.at[1, pl.ds(e0, CH)], col1)

                def seg(j, _):
                    for j2 in range(SUB // 16):
                        o = j * SUB + j2 * 16
                        r = row1[pl.ds(o, 16)]
                        gidx[j, pl.ds(j2 * 16, 16)] = r * NCB + cbi
                        sidx[j, pl.ds(j2 * 16, 16)] = col1[pl.ds(o, 16)]
                    return 0
                lax.fori_loop(0, NSUB, seg, 0)

                descs = [
                    pltpu.async_copy(z_hbm.at[gidx.at[j]],
                                     gbuf.at[pl.ds(j * SUB, SUB)], sem)
                    for j in range(NSUB)
                ]
                for d in descs:
                    d.wait()
                for j in range(NSUB):
                    pltpu.sync_copy(gbuf.at[pl.ds(j * SUB, SUB)],
                                    acc.at[sidx.at[j]], add=True)
                return 0
            lax.fori_loop(0, NCHUNK, chunk, 0)
            plsc.subcore_barrier()
            pltpu.sync_copy(acc.at[pl.ds(wr0, RPT)],
                            agg_hbm.at[pl.ds(wr0, RPT), cbi])
            plsc.subcore_barrier()
            return 0
        lax.fori_loop(0, NCB // 2, one_pass, 0)

    return scat(z2d, ei)


def _sc_scatter_narrow(zin, ei, N, Npad, Epad):
    RACC = Npad
    EPW = Epad // 32
    NCHUNK = EPW // CH
    RPT = Npad // 16
    RZT = RPT
    mesh = plsc.VectorSubcoreMesh(core_axis_name="c", subcore_axis_name="s")

    @functools.partial(
        pl.kernel,
        out_type=jax.ShapeDtypeStruct((2, Npad, CB), jnp.float32),
        mesh=mesh,
        compiler_params=pltpu.CompilerParams(use_tc_tiling_on_sc=False),
        scratch_types=[
            pltpu.VMEM((CH,), jnp.int32),
            pltpu.VMEM((CH,), jnp.int32),
            pltpu.VMEM((NSUB, SUB), jnp.int32),
            pltpu.VMEM((NSUB, SUB), jnp.int32),
            pltpu.VMEM((CH, CB), jnp.float32),
            pltpu.VMEM_SHARED((RACC, CB), jnp.float32),
            pltpu.SemaphoreType.DMA,
        ],
    )
    def scat(z_hbm, ei_hbm, agg_hbm, row1, col1, gidx, sidx, gbuf,
             acc, sem):
        c = lax.axis_index("c")
        s = lax.axis_index("s")

        zr0 = s * RZT
        wr0 = s * RPT
        e_base = (c * 16 + s) * EPW

        def zrow(i, _):
            gbuf[i, :] = jnp.zeros((CB,), jnp.float32)
            return 0
        lax.fori_loop(0, CH, zrow, 0)
        off = 0
        while off < RZT:
            sz = min(CH, RZT - off)
            pltpu.sync_copy(gbuf.at[pl.ds(0, sz)],
                            acc.at[pl.ds(zr0 + off, sz)])
            off += sz
        plsc.subcore_barrier()

        def chunk(g, _):
            e0 = pl.multiple_of(e_base + g * CH, 8)
            pltpu.sync_copy(ei_hbm.at[0, pl.ds(e0, CH)], row1)
            pltpu.sync_copy(ei_hbm.at[1, pl.ds(e0, CH)], col1)

            def seg(j, _):
                for j2 in range(SUB // 16):
                    o = j * SUB + j2 * 16
                    gidx[j, pl.ds(j2 * 16, 16)] = row1[pl.ds(o, 16)]
                    sidx[j, pl.ds(j2 * 16, 16)] = col1[pl.ds(o, 16)]
                return 0
            lax.fori_loop(0, NSUB, seg, 0)

            descs = [
                pltpu.async_copy(z_hbm.at[gidx.at[j]],
                                 gbuf.at[pl.ds(j * SUB, SUB)], sem)
                for j in range(NSUB)
            ]
            for d in descs:
                d.wait()
            for j in range(NSUB):
                pltpu.sync_copy(gbuf.at[pl.ds(j * SUB, SUB)],
                                acc.at[sidx.at[j]], add=True)
            return 0
        lax.fori_loop(0, NCHUNK, chunk, 0)
        plsc.subcore_barrier()
        pltpu.sync_copy(acc.at[pl.ds(wr0, RPT)],
                        agg_hbm.at[c, pl.ds(wr0, RPT)])

    return scat(zin, ei)


def _omegas():
    return (2.0 * math.pi / K) * lax.broadcasted_iota(
        jnp.int32, (1, K), 1).astype(jnp.float32)


def _tc_phases(rw3, N, NB, Bn):
    def body(rw_ref, zin_ref):
        ph = rw_ref[0, 0, :][:, None] * _omegas()
        zin_ref[...] = jnp.concatenate([jnp.cos(ph), jnp.sin(ph)], axis=1)

    return pl.pallas_call(
        body,
        grid=(NB,),
        in_specs=[pl.BlockSpec((1, 1, Bn), lambda i: (i, 0, 0))],
        out_specs=pl.BlockSpec((Bn, 2 * K), lambda i: (i, 0)),
        out_shape=jax.ShapeDtypeStruct((N, 2 * K), jnp.float32),
    )(rw3)


def _full(shape):
    zeros = (0,) * len(shape)
    return pl.BlockSpec(shape, lambda i, z=zeros: z)


def _tc_layer1(rw3, agg0, epsp, w1r, w2r, b1r, b2r, w1i, w2i, b1i, b2i,
               N, NB, Bn):
    def body(rw_ref, agg_ref, eps_ref, w1r_ref, w2r_ref, b1r_ref, b2r_ref,
             w1i_ref, w2i_ref, b1i_ref, b2i_ref, zr_ref, zi_ref):
        ph = rw_ref[0, 0, :][:, None] * _omegas()
        a = agg_ref[0] + agg_ref[1]
        ev = eps_ref[0, 0]
        outr = ev * jnp.cos(ph) + a[:, :K]
        outi = ev * jnp.sin(ph) + a[:, K:]
        hr = jnp.maximum(
            jnp.dot(outr, w1r_ref[...], preferred_element_type=jnp.float32)
            + b1r_ref[...], 0.0)
        zr_ref[...] = jnp.dot(
            hr, w2r_ref[...], preferred_element_type=jnp.float32) + b2r_ref[...]
        hi = jnp.maximum(
            jnp.dot(outi, w1i_ref[...], preferred_element_type=jnp.float32)
            + b1i_ref[...], 0.0)
        zi_ref[...] = jnp.dot(
            hi, w2i_ref[...], preferred_element_type=jnp.float32) + b2i_ref[...]

    return pl.pallas_call(
        body,
        grid=(NB,),
        in_specs=[
            pl.BlockSpec((1, 1, Bn), lambda i: (i, 0, 0)),
            pl.BlockSpec((2, Bn, CB), lambda i: (0, i, 0)),
            _full((1, 1)),
            _full((K, DF)), _full((DF, DF)), _full((1, DF)), _full((1, DF)),
            _full((K, DF)), _full((DF, DF)), _full((1, DF)), _full((1, DF)),
        ],
        out_specs=[pl.BlockSpec((Bn, DF), lambda i: (i, 0))] * 2,
        out_shape=[jax.ShapeDtypeStruct((N, DF), jnp.float32)] * 2,
    )(rw3, agg0, epsp, w1r, w2r, b1r, b2r, w1i, w2i, b1i, b2i)


def _tc_mid(zr, zi, aggr, aggi, epsp, w1r, w2r, b1r, b2r, w1i, w2i, b1i, b2i,
            N, NB, Bn):
    def body(zr_ref, zi_ref, ar_ref, ai_ref, eps_ref, w1r_ref, w2r_ref,
             b1r_ref, b2r_ref, w1i_ref, w2i_ref, b1i_ref, b2i_ref,
             or_ref, oi_ref):
        ev = eps_ref[0, 0]
        outr = ev * zr_ref[...] + ar_ref[...]
        outi = ev * zi_ref[...] + ai_ref[...]
        hr = jnp.maximum(
            jnp.dot(outr, w1r_ref[...], preferred_element_type=jnp.float32)
            + b1r_ref[...], 0.0)
        or_ref[...] = jnp.dot(
            hr, w2r_ref[...], preferred_element_type=jnp.float32) + b2r_ref[...]
        hi = jnp.maximum(
            jnp.dot(outi, w1i_ref[...], preferred_element_type=jnp.float32)
            + b1i_ref[...], 0.0)
        oi_ref[...] = jnp.dot(
            hi, w2i_ref[...], preferred_element_type=jnp.float32) + b2i_ref[...]

    blk = pl.BlockSpec((Bn, DF), lambda i: (i, 0))
    return pl.pallas_call(
        body,
        grid=(NB,),
        in_specs=[blk, blk, blk, blk, _full((1, 1)),
                  _full((DF, DF)), _full((DF, DF)), _full((1, DF)),
                  _full((1, DF)),
                  _full((DF, DF)), _full((DF, DF)), _full((1, DF)),
                  _full((1, DF))],
        out_specs=[blk, blk],
        out_shape=[jax.ShapeDtypeStruct((N, DF), jnp.float32)] * 2,
    )(zr, zi, aggr, aggi, epsp, w1r, w2r, b1r, b2r, w1i, w2i, b1i, b2i)


def _tc_final(zr, zi, aggr, aggi, epsp, w1r, w2r, b1r, b2r, w1i, w2i,
              b1i, b2i, batch3, wc1, bc1, wc2, bc2, N, NB, Bn):
    def body(zr_ref, zi_ref, ar_ref, ai_ref, eps_ref, w1r_ref, w2r_ref,
             b1r_ref, b2r_ref, w1i_ref, w2i_ref, b1i_ref, b2i_ref,
             batch_ref, wc1_ref, bc1_ref, wc2_ref, bc2_ref, out_ref, pooled):
        i = pl.program_id(0)

        @pl.when(i == 0)
        def _():
            pooled[...] = jnp.zeros_like(pooled)

        ev = eps_ref[0, 0]
        outr = ev * zr_ref[...] + ar_ref[...]
        outi = ev * zi_ref[...] + ai_ref[...]
        hr = jnp.maximum(
            jnp.dot(outr, w1r_ref[...], preferred_element_type=jnp.float32)
            + b1r_ref[...], 0.0)
        z3r = jnp.dot(
            hr, w2r_ref[...], preferred_element_type=jnp.float32) + b2r_ref[...]
        hi = jnp.maximum(
            jnp.dot(outi, w1i_ref[...], preferred_element_type=jnp.float32)
            + b1i_ref[...], 0.0)
        z3i = jnp.dot(
            hi, w2i_ref[...], preferred_element_type=jnp.float32) + b2i_ref[...]
        mag = jnp.sqrt(z3r * z3r + z3i * z3i + 1e-08)

        bb = batch_ref[0, 0, :]
        oh = (bb[:, None] == lax.broadcasted_iota(jnp.int32, (Bn, NG), 1)
              ).astype(jnp.float32)
        pooled[...] += lax.dot_general(
            oh, mag, (((0,), (0,)), ((), ())),
            preferred_element_type=jnp.float32)

        @pl.when(i == NB - 1)
        def _():
            h = jnp.maximum(
                jnp.dot(pooled[...], wc1_ref[...],
                        preferred_element_type=jnp.float32) + bc1_ref[...],
                0.0)
            out_ref[...] = jnp.dot(
                h, wc2_ref[...], preferred_element_type=jnp.float32
            ) + bc2_ref[...]

    blk = pl.BlockSpec((Bn, DF), lambda i: (i, 0))
    return pl.pallas_call(
        body,
        grid=(NB,),
        in_specs=[blk, blk, blk, blk, _full((1, 1)),
                  _full((DF, DF)), _full((DF, DF)), _full((1, DF)),
                  _full((1, DF)),
                  _full((DF, DF)), _full((DF, DF)), _full((1, DF)),
                  _full((1, DF)),
                  pl.BlockSpec((1, 1, Bn), lambda i: (i, 0, 0)),
                  _full((DF, HID)), _full((1, HID)), _full((HID, 2)),
                  _full((1, 2))],
        out_specs=pl.BlockSpec((NG, 2), lambda i: (0, 0)),
        out_shape=jax.ShapeDtypeStruct((NG, 2), jnp.float32),
        scratch_shapes=[pltpu.VMEM((NG, DF), jnp.float32)],
    )(zr, zi, aggr, aggi, epsp, w1r, w2r, b1r, b2r, w1i, w2i, b1i, b2i,
      batch3, wc1, bc1, wc2, bc2)


def _block_diag(w):
    return jnp.kron(jnp.eye(K, dtype=jnp.float32), w)


def _pick_bn(N):
    for bn in (2000, 2048, 1024, 512, 256, 128, 64, 32, 16, 8):
        if N % bn == 0:
            return bn
    raise ValueError(f"N={N} has no supported block size")


def kernel(rw_t3, params, edge_index, batch, num_graphs):
    N = rw_t3.shape[0]
    E = edge_index.shape[1]
    assert N % 16 == 0
    Bn = _pick_bn(N)
    NB = N // Bn
    Npad = 16 * (-(-(N // 16) // 8) * 8)

    unit = 32 * CH
    Epad = -(-E // unit) * unit
    if Epad > E:
        pad = jnp.concatenate(
            [jnp.zeros((1, Epad - E), jnp.int32),
             jnp.full((1, Epad - E), N, jnp.int32)], axis=0)
        ei = jnp.concatenate([edge_index, pad], axis=1)
    else:
        ei = edge_index

    rw3 = rw_t3.reshape(NB, 1, Bn)
    batch3 = batch.reshape(NB, 1, Bn)

    lw = []
    for lp in params['layers']:
        lw.append(dict(
            epsp=(1.0 + lp['eps']).reshape(1, 1),
            w1r=_block_diag(lp['W1r']), w2r=_block_diag(lp['W2r']),
            b1r=jnp.tile(lp['b1r'], K)[None, :],
            b2r=jnp.tile(lp['b2r'], K)[None, :],
            w1i=_block_diag(lp['W1i']), w2i=_block_diag(lp['W2i']),
            b1i=jnp.tile(lp['b1i'], K)[None, :],
            b2i=jnp.tile(lp['b2i'], K)[None, :],
        ))

    zin = _tc_phases(rw3, N, NB, Bn)
    agg0 = _sc_scatter_narrow(zin, ei, N, Npad, Epad)
    l = lw[0]
    z1r, z1i = _tc_layer1(rw3, agg0, l['epsp'], l['w1r'], l['w2r'], l['b1r'],
                          l['b2r'], l['w1i'], l['w2i'], l['b1i'], l['b2i'],
                          N, NB, Bn)

    agg1r = _sc_scatter_wide(z1r.reshape(N * NCB, CB), ei, N, Npad, Epad)
    agg1i = _sc_scatter_wide(z1i.reshape(N * NCB, CB), ei, N, Npad, Epad)
    l = lw[1]
    z2r, z2i = _tc_mid(z1r, z1i, agg1r.reshape(Npad, DF),
                       agg1i.reshape(Npad, DF),
                       l['epsp'], l['w1r'], l['w2r'], l['b1r'], l['b2r'],
                       l['w1i'], l['w2i'], l['b1i'], l['b2i'], N, NB, Bn)

    agg2r = _sc_scatter_wide(z2r.reshape(N * NCB, CB), ei, N, Npad, Epad)
    agg2i = _sc_scatter_wide(z2i.reshape(N * NCB, CB), ei, N, Npad, Epad)
    l = lw[2]
    return _tc_final(z2r, z2i, agg2r.reshape(Npad, DF),
                     agg2i.reshape(Npad, DF),
                     l['epsp'], l['w1r'], l['w2r'], l['b1r'], l['b2r'],
                     l['w1i'], l['w2i'], l['b1i'], l['b2i'], batch3,
                     params['Wc1'], params['bc1'][None, :],
                     params['Wc2'], params['bc2'][None, :], N, NB, Bn)


# speedup vs baseline: 23.4750x; 1.3835x over previous
import functools
import math

import jax
import jax.numpy as jnp
from jax import lax
from jax.experimental import pallas as pl
from jax.experimental.pallas import tpu as pltpu
from jax.experimental.pallas import tpu_sc as plsc

K = 8
HID = 32
DF = K * HID
CB = 16
NCB = DF // CB
CH = 512
SUB = 128
NSUB = CH // SUB
NG = 64


def _pipe_accumulate(ei_hbm, z_hbm, acc, B0, B1, e_base, epw, dummy_col,
                     gxf):
    NFULL = epw // CH
    REM = epw - NFULL * CH
    NPAIR = NFULL // 2
    LEFT = NFULL - 2 * NPAIR
    assert NPAIR >= 1 and REM % 16 == 0

    def e0_of(g):
        return pl.multiple_of(e_base + g * CH, 8)

    def gslice(B, j):
        return B[4].at[pl.ds(j * SUB, SUB)]

    def fire_idx(B, g):
        e0 = e0_of(g)
        pltpu.async_copy(ei_hbm.at[0, pl.ds(e0, CH)], B[0], B[5])
        pltpu.async_copy(ei_hbm.at[1, pl.ds(e0, CH)], B[1], B[5])

    def wait_idx(B, g):
        e0 = e0_of(g)
        pltpu.make_async_copy(ei_hbm.at[0, pl.ds(e0, CH)], B[0], B[5]).wait()
        pltpu.make_async_copy(ei_hbm.at[1, pl.ds(e0, CH)], B[1], B[5]).wait()

    def drain_scat(B, n=NSUB):
        for j in range(n):
            pltpu.make_async_copy(gslice(B, j), acc.at[B[3].at[j]],
                                  B[7]).wait()

    def build(B):
        def seg(j, _):
            for j2 in range(SUB // 16):
                o = j * SUB + j2 * 16
                B[2][j, pl.ds(j2 * 16, 16)] = gxf(B[0][pl.ds(o, 16)])
                B[3][j, pl.ds(j2 * 16, 16)] = B[1][pl.ds(o, 16)]
            return 0
        lax.fori_loop(0, NSUB, seg, 0)

    def fire_gathers(B, n=NSUB):
        return [pltpu.async_copy(z_hbm.at[B[2].at[j]], gslice(B, j), B[6])
                for j in range(n)]

    def fire_scats(B, n=NSUB):
        for j in range(n):
            pltpu.async_copy(gslice(B, j), acc.at[B[3].at[j]], B[7],
                             add=True)

    fire_idx(B0, 0)
    fire_idx(B1, 1)

    def pair(g2, _):
        for b, B in ((0, B0), (1, B1)):
            ga = 2 * g2 + b
            wait_idx(B, ga)

            @pl.when(g2 > 0)
            def _():
                drain_scat(B)
            build(B)
            gd = fire_gathers(B)

            @pl.when(ga + 2 < NFULL)
            def _():
                fire_idx(B, ga + 2)
            for d in gd:
                d.wait()
            fire_scats(B)
        return 0
    lax.fori_loop(0, NPAIR, pair, 0)

    out = {0: NSUB, 1: NSUB}
    if LEFT:
        gl = 2 * NPAIR
        wait_idx(B0, gl)
        drain_scat(B0)
        build(B0)
        gd = fire_gathers(B0)
        for d in gd:
            d.wait()
        fire_scats(B0)
        out[0] = NSUB
    if REM:
        br, B = (0, B0) if LEFT == 0 else (1, B1)
        drain_scat(B, out[br])
        e0 = e0_of(NFULL)
        pltpu.sync_copy(ei_hbm.at[0, pl.ds(e0, REM)],
                        B[0].at[pl.ds(0, REM)])
        pltpu.sync_copy(ei_hbm.at[1, pl.ds(e0, REM)],
                        B[1].at[pl.ds(0, REM)])
        nsr = -(-REM // SUB)
        for j in range(nsr):
            for j2 in range(SUB // 16):
                o = j * SUB + j2 * 16
                if o < REM:
                    B[2][j, pl.ds(j2 * 16, 16)] = gxf(B[0][pl.ds(o, 16)])
                    B[3][j, pl.ds(j2 * 16, 16)] = B[1][pl.ds(o, 16)]
                else:
                    B[2][j, pl.ds(j2 * 16, 16)] = jnp.zeros((16,), jnp.int32)
                    B[3][j, pl.ds(j2 * 16, 16)] = jnp.full(
                        (16,), dummy_col, jnp.int32)
        gd = fire_gathers(B, nsr)
        for d in gd:
            d.wait()
        fire_scats(B, nsr)
        out[br] = nsr
    drain_scat(B0, out[0])
    drain_scat(B1, out[1])


_SC_SCRATCH = None


def _sc_scratch_types(RACC):
    buf = [
        pltpu.VMEM((CH,), jnp.int32),
        pltpu.VMEM((CH,), jnp.int32),
        pltpu.VMEM((NSUB, SUB), jnp.int32),
        pltpu.VMEM((NSUB, SUB), jnp.int32),
        pltpu.VMEM((CH, CB), jnp.float32),
    ]
    sems = [pltpu.SemaphoreType.DMA] * 3
    return (buf + buf
            + [pltpu.VMEM_SHARED((RACC, CB), jnp.float32)]
            + sems + sems)


def _zero_acc(gbuf, acc, zr0, RZT):
    def zrow(i, _):
        gbuf[i, :] = jnp.zeros((CB,), jnp.float32)
        return 0
    lax.fori_loop(0, CH, zrow, 0)
    off = 0
    while off < RZT:
        sz = min(CH, RZT - off)
        pltpu.sync_copy(gbuf.at[pl.ds(0, sz)],
                        acc.at[pl.ds(zr0 + off, sz)])
        off += sz


def _sc_scatter_wide(z2d, ei, N, Npad, E):
    RACC = Npad
    EPW = E // 16
    RPT = Npad // 16
    mesh = plsc.VectorSubcoreMesh(core_axis_name="c", subcore_axis_name="s")

    @functools.partial(
        pl.kernel,
        out_type=jax.ShapeDtypeStruct((Npad, NCB, CB), jnp.float32),
        mesh=mesh,
        compiler_params=pltpu.CompilerParams(use_tc_tiling_on_sc=False),
        scratch_types=_sc_scratch_types(RACC),
    )
    def scat(z_hbm, ei_hbm, agg_hbm,
             r0b, c0b, g0b, s0b, f0b, r1b, c1b, g1b, s1b, f1b,
             acc, si0, sg0, ss0, si1, sg1, ss1):
        c = lax.axis_index("c")
        s = lax.axis_index("s")
        B0 = (r0b, c0b, g0b, s0b, f0b, si0, sg0, ss0)
        B1 = (r1b, c1b, g1b, s1b, f1b, si1, sg1, ss1)
        zr0 = s * RPT
        e_base = s * EPW

        def one_pass(p, _):
            cbi = 2 * p + c
            _zero_acc(f0b, acc, zr0, RPT)
            plsc.subcore_barrier()
            _pipe_accumulate(ei_hbm, z_hbm, acc, B0, B1, e_base, EPW, N,
                             lambda v: v * NCB + cbi)
            plsc.subcore_barrier()
            pltpu.sync_copy(acc.at[pl.ds(zr0, RPT)],
                            agg_hbm.at[pl.ds(zr0, RPT), cbi])
            plsc.subcore_barrier()
            return 0
        lax.fori_loop(0, NCB // 2, one_pass, 0)

    return scat(z2d, ei)


def _sc_scatter_narrow(zin, ei, N, Npad, E):
    RACC = Npad
    EPW = E // 32
    RPT = Npad // 16
    mesh = plsc.VectorSubcoreMesh(core_axis_name="c", subcore_axis_name="s")

    @functools.partial(
        pl.kernel,
        out_type=jax.ShapeDtypeStruct((2, Npad, CB), jnp.float32),
        mesh=mesh,
        compiler_params=pltpu.CompilerParams(use_tc_tiling_on_sc=False),
        scratch_types=_sc_scratch_types(RACC),
    )
    def scat(z_hbm, ei_hbm, agg_hbm,
             r0b, c0b, g0b, s0b, f0b, r1b, c1b, g1b, s1b, f1b,
             acc, si0, sg0, ss0, si1, sg1, ss1):
        c = lax.axis_index("c")
        s = lax.axis_index("s")
        B0 = (r0b, c0b, g0b, s0b, f0b, si0, sg0, ss0)
        B1 = (r1b, c1b, g1b, s1b, f1b, si1, sg1, ss1)
        zr0 = s * RPT
        e_base = (c * 16 + s) * EPW

        _zero_acc(f0b, acc, zr0, RPT)
        plsc.subcore_barrier()
        _pipe_accumulate(ei_hbm, z_hbm, acc, B0, B1, e_base, EPW, N,
                         lambda v: v)
        plsc.subcore_barrier()
        pltpu.sync_copy(acc.at[pl.ds(zr0, RPT)],
                        agg_hbm.at[c, pl.ds(zr0, RPT)])

    return scat(zin, ei)


def _omegas():
    return (2.0 * math.pi / K) * lax.broadcasted_iota(
        jnp.int32, (1, K), 1).astype(jnp.float32)


def _tc_phases(rw3, N, NB, Bn):
    def body(rw_ref, zin_ref):
        ph = rw_ref[0, 0, :][:, None] * _omegas()
        zin_ref[...] = jnp.concatenate([jnp.cos(ph), jnp.sin(ph)], axis=1)

    return pl.pallas_call(
        body,
        grid=(NB,),
        in_specs=[pl.BlockSpec((1, 1, Bn), lambda i: (i, 0, 0))],
        out_specs=pl.BlockSpec((Bn, 2 * K), lambda i: (i, 0)),
        out_shape=jax.ShapeDtypeStruct((N, 2 * K), jnp.float32),
    )(rw3)


def _full(shape):
    zeros = (0,) * len(shape)
    return pl.BlockSpec(shape, lambda i, z=zeros: z)


def _tc_layer1(rw3, agg0, epsp, w1r, w2r, b1r, b2r, w1i, w2i, b1i, b2i,
               N, NB, Bn):
    def body(rw_ref, agg_ref, eps_ref, w1r_ref, w2r_ref, b1r_ref, b2r_ref,
             w1i_ref, w2i_ref, b1i_ref, b2i_ref, zr_ref, zi_ref):
        ph = rw_ref[0, 0, :][:, None] * _omegas()
        a = agg_ref[0] + agg_ref[1]
        ev = eps_ref[0, 0]
        outr = ev * jnp.cos(ph) + a[:, :K]
        outi = ev * jnp.sin(ph) + a[:, K:]
        hr = jnp.maximum(
            jnp.dot(outr, w1r_ref[...], preferred_element_type=jnp.float32)
            + b1r_ref[...], 0.0)
        zr_ref[...] = jnp.dot(
            hr, w2r_ref[...], preferred_element_type=jnp.float32) + b2r_ref[...]
        hi = jnp.maximum(
            jnp.dot(outi, w1i_ref[...], preferred_element_type=jnp.float32)
            + b1i_ref[...], 0.0)
        zi_ref[...] = jnp.dot(
            hi, w2i_ref[...], preferred_element_type=jnp.float32) + b2i_ref[...]

    return pl.pallas_call(
        body,
        grid=(NB,),
        in_specs=[
            pl.BlockSpec((1, 1, Bn), lambda i: (i, 0, 0)),
            pl.BlockSpec((2, Bn, CB), lambda i: (0, i, 0)),
            _full((1, 1)),
            _full((K, DF)), _full((DF, DF)), _full((1, DF)), _full((1, DF)),
            _full((K, DF)), _full((DF, DF)), _full((1, DF)), _full((1, DF)),
        ],
        out_specs=[pl.BlockSpec((Bn, DF), lambda i: (i, 0))] * 2,
        out_shape=[jax.ShapeDtypeStruct((N, DF), jnp.float32)] * 2,
    )(rw3, agg0, epsp, w1r, w2r, b1r, b2r, w1i, w2i, b1i, b2i)


def _tc_mid(zr, zi, aggr, aggi, epsp, w1r, w2r, b1r, b2r, w1i, w2i, b1i, b2i,
            N, NB, Bn):
    def body(zr_ref, zi_ref, ar_ref, ai_ref, eps_ref, w1r_ref, w2r_ref,
             b1r_ref, b2r_ref, w1i_ref, w2i_ref, b1i_ref, b2i_ref,
             or_ref, oi_ref):
        ev = eps_ref[0, 0]
        outr = ev * zr_ref[...] + ar_ref[...]
        outi = ev * zi_ref[...] + ai_ref[...]
        hr = jnp.maximum(
            jnp.dot(outr, w1r_ref[...], preferred_element_type=jnp.float32)
            + b1r_ref[...], 0.0)
        or_ref[...] = jnp.dot(
            hr, w2r_ref[...], preferred_element_type=jnp.float32) + b2r_ref[...]
        hi = jnp.maximum(
            jnp.dot(outi, w1i_ref[...], preferred_element_type=jnp.float32)
            + b1i_ref[...], 0.0)
        oi_ref[...] = jnp.dot(
            hi, w2i_ref[...], preferred_element_type=jnp.float32) + b2i_ref[...]

    blk = pl.BlockSpec((Bn, DF), lambda i: (i, 0))
    return pl.pallas_call(
        body,
        grid=(NB,),
        in_specs=[blk, blk, blk, blk, _full((1, 1)),
                  _full((DF, DF)), _full((DF, DF)), _full((1, DF)),
                  _full((1, DF)),
                  _full((DF, DF)), _full((DF, DF)), _full((1, DF)),
                  _full((1, DF))],
        out_specs=[blk, blk],
        out_shape=[jax.ShapeDtypeStruct((N, DF), jnp.float32)] * 2,
    )(zr, zi, aggr, aggi, epsp, w1r, w2r, b1r, b2r, w1i, w2i, b1i, b2i)


def _tc_final(zr, zi, aggr, aggi, epsp, w1r, w2r, b1r, b2r, w1i, w2i,
              b1i, b2i, batch3, wc1, bc1, wc2, bc2, N, NB, Bn):
    def body(zr_ref, zi_ref, ar_ref, ai_ref, eps_ref, w1r_ref, w2r_ref,
             b1r_ref, b2r_ref, w1i_ref, w2i_ref, b1i_ref, b2i_ref,
             batch_ref, wc1_ref, bc1_ref, wc2_ref, bc2_ref, out_ref, pooled):
        i = pl.program_id(0)

        @pl.when(i == 0)
        def _():
            pooled[...] = jnp.zeros_like(pooled)

        ev = eps_ref[0, 0]
        outr = ev * zr_ref[...] + ar_ref[...]
        outi = ev * zi_ref[...] + ai_ref[...]
        hr = jnp.maximum(
            jnp.dot(outr, w1r_ref[...], preferred_element_type=jnp.float32)
            + b1r_ref[...], 0.0)
        z3r = jnp.dot(
            hr, w2r_ref[...], preferred_element_type=jnp.float32) + b2r_ref[...]
        hi = jnp.maximum(
            jnp.dot(outi, w1i_ref[...], preferred_element_type=jnp.float32)
            + b1i_ref[...], 0.0)
        z3i = jnp.dot(
            hi, w2i_ref[...], preferred_element_type=jnp.float32) + b2i_ref[...]
        mag = jnp.sqrt(z3r * z3r + z3i * z3i + 1e-08)

        bb = batch_ref[0, 0, :]
        oh = (bb[:, None] == lax.broadcasted_iota(jnp.int32, (Bn, NG), 1)
              ).astype(jnp.float32)
        pooled[...] += lax.dot_general(
            oh, mag, (((0,), (0,)), ((), ())),
            preferred_element_type=jnp.float32)

        @pl.when(i == NB - 1)
        def _():
            h = jnp.maximum(
                jnp.dot(pooled[...], wc1_ref[...],
                        preferred_element_type=jnp.float32) + bc1_ref[...],
                0.0)
            out_ref[...] = jnp.dot(
                h, wc2_ref[...], preferred_element_type=jnp.float32
            ) + bc2_ref[...]

    blk = pl.BlockSpec((Bn, DF), lambda i: (i, 0))
    return pl.pallas_call(
        body,
        grid=(NB,),
        in_specs=[blk, blk, blk, blk, _full((1, 1)),
                  _full((DF, DF)), _full((DF, DF)), _full((1, DF)),
                  _full((1, DF)),
                  _full((DF, DF)), _full((DF, DF)), _full((1, DF)),
                  _full((1, DF)),
                  pl.BlockSpec((1, 1, Bn), lambda i: (i, 0, 0)),
                  _full((DF, HID)), _full((1, HID)), _full((HID, 2)),
                  _full((1, 2))],
        out_specs=pl.BlockSpec((NG, 2), lambda i: (0, 0)),
        out_shape=jax.ShapeDtypeStruct((NG, 2), jnp.float32),
        scratch_shapes=[pltpu.VMEM((NG, DF), jnp.float32)],
    )(zr, zi, aggr, aggi, epsp, w1r, w2r, b1r, b2r, w1i, w2i, b1i, b2i,
      batch3, wc1, bc1, wc2, bc2)


def _block_diag(w):
    return jnp.kron(jnp.eye(K, dtype=jnp.float32), w)


def _pick_bn(N):
    for bn in (2000, 2048, 1024, 512, 256, 128, 64, 32, 16, 8):
        if N % bn == 0:
            return bn
    raise ValueError(f"N={N} has no supported block size")


def kernel(rw_t3, params, edge_index, batch, num_graphs):
    N = rw_t3.shape[0]
    E = edge_index.shape[1]
    assert N % 16 == 0
    Bn = _pick_bn(N)
    NB = N // Bn
    Npad = 16 * (-(-(N // 16) // 8) * 8)

    assert E % 512 == 0
    ei = edge_index

    rw3 = rw_t3.reshape(NB, 1, Bn)
    batch3 = batch.reshape(NB, 1, Bn)

    lw = []
    for lp in params['layers']:
        lw.append(dict(
            epsp=(1.0 + lp['eps']).reshape(1, 1),
            w1r=_block_diag(lp['W1r']), w2r=_block_diag(lp['W2r']),
            b1r=jnp.tile(lp['b1r'], K)[None, :],
            b2r=jnp.tile(lp['b2r'], K)[None, :],
            w1i=_block_diag(lp['W1i']), w2i=_block_diag(lp['W2i']),
            b1i=jnp.tile(lp['b1i'], K)[None, :],
            b2i=jnp.tile(lp['b2i'], K)[None, :],
        ))

    zin = _tc_phases(rw3, N, NB, Bn)
    agg0 = _sc_scatter_narrow(zin, ei, N, Npad, E)
    l = lw[0]
    z1r, z1i = _tc_layer1(rw3, agg0, l['epsp'], l['w1r'], l['w2r'], l['b1r'],
                          l['b2r'], l['w1i'], l['w2i'], l['b1i'], l['b2i'],
                          N, NB, Bn)

    agg1r = _sc_scatter_wide(z1r.reshape(N * NCB, CB), ei, N, Npad, E)
    agg1i = _sc_scatter_wide(z1i.reshape(N * NCB, CB), ei, N, Npad, E)
    l = lw[1]
    z2r, z2i = _tc_mid(z1r, z1i, agg1r.reshape(Npad, DF),
                       agg1i.reshape(Npad, DF),
                       l['epsp'], l['w1r'], l['w2r'], l['b1r'], l['b2r'],
                       l['w1i'], l['w2i'], l['b1i'], l['b2i'], N, NB, Bn)

    agg2r = _sc_scatter_wide(z2r.reshape(N * NCB, CB), ei, N, Npad, E)
    agg2i = _sc_scatter_wide(z2i.reshape(N * NCB, CB), ei, N, Npad, E)
    l = lw[2]
    return _tc_final(z2r, z2i, agg2r.reshape(Npad, DF),
                     agg2i.reshape(Npad, DF),
                     l['epsp'], l['w1r'], l['w2r'], l['b1r'], l['b2r'],
                     l['w1i'], l['w2i'], l['b1i'], l['b2i'], batch3,
                     params['Wc1'], params['bc1'][None, :],
                     params['Wc2'], params['bc2'][None, :], N, NB, Bn)


# speedup vs baseline: 28.3416x; 1.2073x over previous
import functools
import math

import jax
import jax.numpy as jnp
from jax import lax
from jax.experimental import pallas as pl
from jax.experimental.pallas import tpu as pltpu
from jax.experimental.pallas import tpu_sc as plsc

K = 8
HID = 32
DF = K * HID
CB = 16
NCB = DF // CB
CH = 512
SUB = 128
NSUB = CH // SUB
NG = 64


def _pipe_accumulate(ei_hbm, z_hbm, acc, B0, B1, e_base, epw, dummy_col,
                     gxf):
    NFULL = epw // CH
    REM = epw - NFULL * CH
    NPAIR = NFULL // 2
    LEFT = NFULL - 2 * NPAIR
    assert NPAIR >= 1 and REM % 16 == 0

    def e0_of(g):
        return pl.multiple_of(e_base + g * CH, 8)

    def gslice(B, j):
        return B[4].at[pl.ds(j * SUB, SUB)]

    def fire_idx(B, g):
        e0 = e0_of(g)
        pltpu.async_copy(ei_hbm.at[0, pl.ds(e0, CH)], B[0], B[5])
        pltpu.async_copy(ei_hbm.at[1, pl.ds(e0, CH)], B[1], B[5])

    def wait_idx(B, g):
        e0 = e0_of(g)
        pltpu.make_async_copy(ei_hbm.at[0, pl.ds(e0, CH)], B[0], B[5]).wait()
        pltpu.make_async_copy(ei_hbm.at[1, pl.ds(e0, CH)], B[1], B[5]).wait()

    def drain_scat(B, n=NSUB):
        for j in range(n):
            pltpu.make_async_copy(gslice(B, j), acc.at[B[3].at[j]],
                                  B[7]).wait()

    def build(B):
        def seg(j, _):
            for j2 in range(SUB // 16):
                o = j * SUB + j2 * 16
                B[2][j, pl.ds(j2 * 16, 16)] = gxf(B[0][pl.ds(o, 16)])
                B[3][j, pl.ds(j2 * 16, 16)] = B[1][pl.ds(o, 16)]
            return 0
        lax.fori_loop(0, NSUB, seg, 0)

    def fire_gathers(B, n=NSUB):
        return [pltpu.async_copy(z_hbm.at[B[2].at[j]], gslice(B, j), B[6])
                for j in range(n)]

    def fire_scats(B, n=NSUB):
        for j in range(n):
            pltpu.async_copy(gslice(B, j), acc.at[B[3].at[j]], B[7],
                             add=True)

    fire_idx(B0, 0)
    fire_idx(B1, 1)

    def pair(g2, _):
        gds = []
        for b, B in ((0, B0), (1, B1)):
            ga = 2 * g2 + b
            wait_idx(B, ga)

            @pl.when(g2 > 0)
            def _():
                drain_scat(B)
            build(B)
            gds.append(fire_gathers(B))

            @pl.when(ga + 2 < NFULL)
            def _():
                fire_idx(B, ga + 2)
        for gd, B in zip(gds, (B0, B1)):
            for d in gd:
                d.wait()
            fire_scats(B)
        return 0
    lax.fori_loop(0, NPAIR, pair, 0)

    out = {0: NSUB, 1: NSUB}
    if LEFT:
        gl = 2 * NPAIR
        wait_idx(B0, gl)
        drain_scat(B0)
        build(B0)
        gd = fire_gathers(B0)
        for d in gd:
            d.wait()
        fire_scats(B0)
        out[0] = NSUB
    if REM:
        br, B = (0, B0) if LEFT == 0 else (1, B1)
        drain_scat(B, out[br])
        e0 = e0_of(NFULL)
        pltpu.sync_copy(ei_hbm.at[0, pl.ds(e0, REM)],
                        B[0].at[pl.ds(0, REM)])
        pltpu.sync_copy(ei_hbm.at[1, pl.ds(e0, REM)],
                        B[1].at[pl.ds(0, REM)])
        nsr = -(-REM // SUB)
        for j in range(nsr):
            for j2 in range(SUB // 16):
                o = j * SUB + j2 * 16
                if o < REM:
                    B[2][j, pl.ds(j2 * 16, 16)] = gxf(B[0][pl.ds(o, 16)])
                    B[3][j, pl.ds(j2 * 16, 16)] = B[1][pl.ds(o, 16)]
                else:
                    B[2][j, pl.ds(j2 * 16, 16)] = jnp.zeros((16,), jnp.int32)
                    B[3][j, pl.ds(j2 * 16, 16)] = jnp.full(
                        (16,), dummy_col, jnp.int32)
        gd = fire_gathers(B, nsr)
        for d in gd:
            d.wait()
        fire_scats(B, nsr)
        out[br] = nsr
    drain_scat(B0, out[0])
    drain_scat(B1, out[1])


_SC_SCRATCH = None


def _sc_scratch_types(RACC):
    buf = [
        pltpu.VMEM((CH,), jnp.int32),
        pltpu.VMEM((CH,), jnp.int32),
        pltpu.VMEM((NSUB, SUB), jnp.int32),
        pltpu.VMEM((NSUB, SUB), jnp.int32),
        pltpu.VMEM((CH, CB), jnp.float32),
    ]
    sems = [pltpu.SemaphoreType.DMA] * 3
    return (buf + buf
            + [pltpu.VMEM_SHARED((RACC, CB), jnp.float32)]
            + sems + sems)


def _zero_acc(gbuf, acc, zr0, RZT):
    def zrow(i, _):
        gbuf[i, :] = jnp.zeros((CB,), jnp.float32)
        return 0
    lax.fori_loop(0, CH, zrow, 0)
    off = 0
    while off < RZT:
        sz = min(CH, RZT - off)
        pltpu.sync_copy(gbuf.at[pl.ds(0, sz)],
                        acc.at[pl.ds(zr0 + off, sz)])
        off += sz


def _sc_scatter_wide(z2d, ei, N, Npad, E):
    RACC = Npad
    EPW = E // 16
    RPT = Npad // 16
    mesh = plsc.VectorSubcoreMesh(core_axis_name="c", subcore_axis_name="s")

    @functools.partial(
        pl.kernel,
        out_type=jax.ShapeDtypeStruct((Npad, NCB, CB), jnp.float32),
        mesh=mesh,
        compiler_params=pltpu.CompilerParams(use_tc_tiling_on_sc=False),
        scratch_types=_sc_scratch_types(RACC),
    )
    def scat(z_hbm, ei_hbm, agg_hbm,
             r0b, c0b, g0b, s0b, f0b, r1b, c1b, g1b, s1b, f1b,
             acc, si0, sg0, ss0, si1, sg1, ss1):
        c = lax.axis_index("c")
        s = lax.axis_index("s")
        B0 = (r0b, c0b, g0b, s0b, f0b, si0, sg0, ss0)
        B1 = (r1b, c1b, g1b, s1b, f1b, si1, sg1, ss1)
        zr0 = s * RPT
        e_base = s * EPW

        def one_pass(p, _):
            cbi = 2 * p + c
            _zero_acc(f0b, acc, zr0, RPT)
            plsc.subcore_barrier()
            _pipe_accumulate(ei_hbm, z_hbm, acc, B0, B1, e_base, EPW, N,
                             lambda v: v * NCB + cbi)
            plsc.subcore_barrier()
            pltpu.sync_copy(acc.at[pl.ds(zr0, RPT)],
                            agg_hbm.at[pl.ds(zr0, RPT), cbi])
            plsc.subcore_barrier()
            return 0
        lax.fori_loop(0, NCB // 2, one_pass, 0)

    return scat(z2d, ei)


def _sc_scatter_narrow(zin, ei, N, Npad, E):
    RACC = Npad
    EPW = E // 32
    RPT = Npad // 16
    mesh = plsc.VectorSubcoreMesh(core_axis_name="c", subcore_axis_name="s")

    @functools.partial(
        pl.kernel,
        out_type=jax.ShapeDtypeStruct((2, Npad, CB), jnp.float32),
        mesh=mesh,
        compiler_params=pltpu.CompilerParams(use_tc_tiling_on_sc=False),
        scratch_types=_sc_scratch_types(RACC),
    )
    def scat(z_hbm, ei_hbm, agg_hbm,
             r0b, c0b, g0b, s0b, f0b, r1b, c1b, g1b, s1b, f1b,
             acc, si0, sg0, ss0, si1, sg1, ss1):
        c = lax.axis_index("c")
        s = lax.axis_index("s")
        B0 = (r0b, c0b, g0b, s0b, f0b, si0, sg0, ss0)
        B1 = (r1b, c1b, g1b, s1b, f1b, si1, sg1, ss1)
        zr0 = s * RPT
        e_base = (c * 16 + s) * EPW

        _zero_acc(f0b, acc, zr0, RPT)
        plsc.subcore_barrier()
        _pipe_accumulate(ei_hbm, z_hbm, acc, B0, B1, e_base, EPW, N,
                         lambda v: v)
        plsc.subcore_barrier()
        pltpu.sync_copy(acc.at[pl.ds(zr0, RPT)],
                        agg_hbm.at[c, pl.ds(zr0, RPT)])

    return scat(zin, ei)


def _omegas():
    return (2.0 * math.pi / K) * lax.broadcasted_iota(
        jnp.int32, (1, K), 1).astype(jnp.float32)


def _tc_phases(rw3, N, NB, Bn):
    def body(rw_ref, zin_ref):
        ph = rw_ref[0, 0, :][:, None] * _omegas()
        zin_ref[...] = jnp.concatenate([jnp.cos(ph), jnp.sin(ph)], axis=1)

    return pl.pallas_call(
        body,
        grid=(NB,),
        in_specs=[pl.BlockSpec((1, 1, Bn), lambda i: (i, 0, 0))],
        out_specs=pl.BlockSpec((Bn, 2 * K), lambda i: (i, 0)),
        out_shape=jax.ShapeDtypeStruct((N, 2 * K), jnp.float32),
    )(rw3)


def _full(shape):
    zeros = (0,) * len(shape)
    return pl.BlockSpec(shape, lambda i, z=zeros: z)


def _tc_layer1(rw3, agg0, epsp, w1r, w2r, b1r, b2r, w1i, w2i, b1i, b2i,
               N, NB, Bn):
    def body(rw_ref, agg_ref, eps_ref, w1r_ref, w2r_ref, b1r_ref, b2r_ref,
             w1i_ref, w2i_ref, b1i_ref, b2i_ref, zr_ref, zi_ref):
        ph = rw_ref[0, 0, :][:, None] * _omegas()
        a = agg_ref[0] + agg_ref[1]
        ev = eps_ref[0, 0]
        outr = ev * jnp.cos(ph) + a[:, :K]
        outi = ev * jnp.sin(ph) + a[:, K:]
        hr = jnp.maximum(
            jnp.dot(outr, w1r_ref[...], preferred_element_type=jnp.float32)
            + b1r_ref[...], 0.0)
        zr_ref[...] = jnp.dot(
            hr, w2r_ref[...], preferred_element_type=jnp.float32) + b2r_ref[...]
        hi = jnp.maximum(
            jnp.dot(outi, w1i_ref[...], preferred_element_type=jnp.float32)
            + b1i_ref[...], 0.0)
        zi_ref[...] = jnp.dot(
            hi, w2i_ref[...], preferred_element_type=jnp.float32) + b2i_ref[...]

    return pl.pallas_call(
        body,
        grid=(NB,),
        in_specs=[
            pl.BlockSpec((1, 1, Bn), lambda i: (i, 0, 0)),
            pl.BlockSpec((2, Bn, CB), lambda i: (0, i, 0)),
            _full((1, 1)),
            _full((K, DF)), _full((DF, DF)), _full((1, DF)), _full((1, DF)),
            _full((K, DF)), _full((DF, DF)), _full((1, DF)), _full((1, DF)),
        ],
        out_specs=[pl.BlockSpec((Bn, DF), lambda i: (i, 0))] * 2,
        out_shape=[jax.ShapeDtypeStruct((N, DF), jnp.float32)] * 2,
    )(rw3, agg0, epsp, w1r, w2r, b1r, b2r, w1i, w2i, b1i, b2i)


def _tc_mid(zr, zi, aggr, aggi, epsp, w1r, w2r, b1r, b2r, w1i, w2i, b1i, b2i,
            N, NB, Bn):
    def body(zr_ref, zi_ref, ar_ref, ai_ref, eps_ref, w1r_ref, w2r_ref,
             b1r_ref, b2r_ref, w1i_ref, w2i_ref, b1i_ref, b2i_ref,
             or_ref, oi_ref):
        ev = eps_ref[0, 0]
        outr = ev * zr_ref[...] + ar_ref[...]
        outi = ev * zi_ref[...] + ai_ref[...]
        hr = jnp.maximum(
            jnp.dot(outr, w1r_ref[...], preferred_element_type=jnp.float32)
            + b1r_ref[...], 0.0)
        or_ref[...] = jnp.dot(
            hr, w2r_ref[...], preferred_element_type=jnp.float32) + b2r_ref[...]
        hi = jnp.maximum(
            jnp.dot(outi, w1i_ref[...], preferred_element_type=jnp.float32)
            + b1i_ref[...], 0.0)
        oi_ref[...] = jnp.dot(
            hi, w2i_ref[...], preferred_element_type=jnp.float32) + b2i_ref[...]

    blk = pl.BlockSpec((Bn, DF), lambda i: (i, 0))
    return pl.pallas_call(
        body,
        grid=(NB,),
        in_specs=[blk, blk, blk, blk, _full((1, 1)),
                  _full((DF, DF)), _full((DF, DF)), _full((1, DF)),
                  _full((1, DF)),
                  _full((DF, DF)), _full((DF, DF)), _full((1, DF)),
                  _full((1, DF))],
        out_specs=[blk, blk],
        out_shape=[jax.ShapeDtypeStruct((N, DF), jnp.float32)] * 2,
    )(zr, zi, aggr, aggi, epsp, w1r, w2r, b1r, b2r, w1i, w2i, b1i, b2i)


def _tc_final(zr, zi, aggr, aggi, epsp, w1r, w2r, b1r, b2r, w1i, w2i,
              b1i, b2i, batch3, wc1, bc1, wc2, bc2, N, NB, Bn):
    def body(zr_ref, zi_ref, ar_ref, ai_ref, eps_ref, w1r_ref, w2r_ref,
             b1r_ref, b2r_ref, w1i_ref, w2i_ref, b1i_ref, b2i_ref,
             batch_ref, wc1_ref, bc1_ref, wc2_ref, bc2_ref, out_ref, pooled):
        i = pl.program_id(0)

        @pl.when(i == 0)
        def _():
            pooled[...] = jnp.zeros_like(pooled)

        ev = eps_ref[0, 0]
        outr = ev * zr_ref[...] + ar_ref[...]
        outi = ev * zi_ref[...] + ai_ref[...]
        hr = jnp.maximum(
            jnp.dot(outr, w1r_ref[...], preferred_element_type=jnp.float32)
            + b1r_ref[...], 0.0)
        z3r = jnp.dot(
            hr, w2r_ref[...], preferred_element_type=jnp.float32) + b2r_ref[...]
        hi = jnp.maximum(
            jnp.dot(outi, w1i_ref[...], preferred_element_type=jnp.float32)
            + b1i_ref[...], 0.0)
        z3i = jnp.dot(
            hi, w2i_ref[...], preferred_element_type=jnp.float32) + b2i_ref[...]
        mag = jnp.sqrt(z3r * z3r + z3i * z3i + 1e-08)

        bb = batch_ref[0, 0, :]
        oh = (bb[:, None] == lax.broadcasted_iota(jnp.int32, (Bn, NG), 1)
              ).astype(jnp.float32)
        pooled[...] += lax.dot_general(
            oh, mag, (((0,), (0,)), ((), ())),
            preferred_element_type=jnp.float32)

        @pl.when(i == NB - 1)
        def _():
            h = jnp.maximum(
                jnp.dot(pooled[...], wc1_ref[...],
                        preferred_element_type=jnp.float32) + bc1_ref[...],
                0.0)
            out_ref[...] = jnp.dot(
                h, wc2_ref[...], preferred_element_type=jnp.float32
            ) + bc2_ref[...]

    blk = pl.BlockSpec((Bn, DF), lambda i: (i, 0))
    return pl.pallas_call(
        body,
        grid=(NB,),
        in_specs=[blk, blk, blk, blk, _full((1, 1)),
                  _full((DF, DF)), _full((DF, DF)), _full((1, DF)),
                  _full((1, DF)),
                  _full((DF, DF)), _full((DF, DF)), _full((1, DF)),
                  _full((1, DF)),
                  pl.BlockSpec((1, 1, Bn), lambda i: (i, 0, 0)),
                  _full((DF, HID)), _full((1, HID)), _full((HID, 2)),
                  _full((1, 2))],
        out_specs=pl.BlockSpec((NG, 2), lambda i: (0, 0)),
        out_shape=jax.ShapeDtypeStruct((NG, 2), jnp.float32),
        scratch_shapes=[pltpu.VMEM((NG, DF), jnp.float32)],
    )(zr, zi, aggr, aggi, epsp, w1r, w2r, b1r, b2r, w1i, w2i, b1i, b2i,
      batch3, wc1, bc1, wc2, bc2)


def _block_diag(w):
    return jnp.kron(jnp.eye(K, dtype=jnp.float32), w)


def _pick_bn(N):
    for bn in (2000, 2048, 1024, 512, 256, 128, 64, 32, 16, 8):
        if N % bn == 0:
            return bn
    raise ValueError(f"N={N} has no supported block size")


def kernel(rw_t3, params, edge_index, batch, num_graphs):
    N = rw_t3.shape[0]
    E = edge_index.shape[1]
    assert N % 16 == 0
    Bn = _pick_bn(N)
    NB = N // Bn
    Npad = 16 * (-(-(N // 16) // 8) * 8)

    assert E % 512 == 0
    ei = edge_index

    rw3 = rw_t3.reshape(NB, 1, Bn)
    batch3 = batch.reshape(NB, 1, Bn)

    lw = []
    for lp in params['layers']:
        lw.append(dict(
            epsp=(1.0 + lp['eps']).reshape(1, 1),
            w1r=_block_diag(lp['W1r']), w2r=_block_diag(lp['W2r']),
            b1r=jnp.tile(lp['b1r'], K)[None, :],
            b2r=jnp.tile(lp['b2r'], K)[None, :],
            w1i=_block_diag(lp['W1i']), w2i=_block_diag(lp['W2i']),
            b1i=jnp.tile(lp['b1i'], K)[None, :],
            b2i=jnp.tile(lp['b2i'], K)[None, :],
        ))

    zin = _tc_phases(rw3, N, NB, Bn)
    agg0 = _sc_scatter_narrow(zin, ei, N, Npad, E)
    l = lw[0]
    z1r, z1i = _tc_layer1(rw3, agg0, l['epsp'], l['w1r'], l['w2r'], l['b1r'],
                          l['b2r'], l['w1i'], l['w2i'], l['b1i'], l['b2i'],
                          N, NB, Bn)

    agg1r = _sc_scatter_wide(z1r.reshape(N * NCB, CB), ei, N, Npad, E)
    agg1i = _sc_scatter_wide(z1i.reshape(N * NCB, CB), ei, N, Npad, E)
    l = lw[1]
    z2r, z2i = _tc_mid(z1r, z1i, agg1r.reshape(Npad, DF),
                       agg1i.reshape(Npad, DF),
                       l['epsp'], l['w1r'], l['w2r'], l['b1r'], l['b2r'],
                       l['w1i'], l['w2i'], l['b1i'], l['b2i'], N, NB, Bn)

    agg2r = _sc_scatter_wide(z2r.reshape(N * NCB, CB), ei, N, Npad, E)
    agg2i = _sc_scatter_wide(z2i.reshape(N * NCB, CB), ei, N, Npad, E)
    l = lw[2]
    return _tc_final(z2r, z2i, agg2r.reshape(Npad, DF),
                     agg2i.reshape(Npad, DF),
                     l['epsp'], l['w1r'], l['w2r'], l['b1r'], l['b2r'],
                     l['w1i'], l['w2i'], l['b1i'], l['b2i'], batch3,
                     params['Wc1'], params['bc1'][None, :],
                     params['Wc2'], params['bc2'][None, :], N, NB, Bn)
